# baseline (speedup 1.0000x reference)
import functools

import jax
import jax.numpy as jnp
from jax import lax
from jax.experimental import pallas as pl
from jax.experimental.pallas import tpu as pltpu
from jax.experimental.pallas import tpu_sc as plsc

D = 128
BE = 4000
BN = 2000

NC = 2
NS = 16
NW = NC * NS
CB = 128


def _silu(v):
    return v * jax.nn.sigmoid(v)


def _ln(v, g, b, eps=1e-5):
    m = jnp.mean(v, axis=-1, keepdims=True)
    c = v - m
    var = jnp.mean(c * c, axis=-1, keepdims=True)
    return c * jax.lax.rsqrt(var + eps) * g + b


def _full(shape):
    return pl.BlockSpec(shape, lambda i: tuple(0 for _ in shape))


def _emb_body(ea_ref, w1_ref, b1_ref, w2_ref, b2_ref, g_ref, bt_ref, out_ref):
    h = _silu(jnp.dot(ea_ref[...], w1_ref[...],
                      preferred_element_type=jnp.float32) + b1_ref[...])
    h = _silu(jnp.dot(h, w2_ref[...],
                      preferred_element_type=jnp.float32) + b2_ref[...])
    out_ref[...] = _ln(h, g_ref[...], bt_ref[...])


def _emb_mlp(ea, w1, b1, w2, b2, g, bt, off_blocks, nblocks):
    ed = ea.shape[1]
    return pl.pallas_call(
        _emb_body,
        grid=(nblocks,),
        in_specs=[
            pl.BlockSpec((BE, ed), lambda i: (i + off_blocks, 0)),
            _full((ed, D)), _full((1, D)), _full((D, D)), _full((1, D)),
            _full((1, D)), _full((1, D)),
        ],
        out_specs=pl.BlockSpec((BE, D), lambda i: (i, 0)),
        out_shape=jax.ShapeDtypeStruct((nblocks * BE, D), jnp.float32),
    )(ea, w1, b1.reshape(1, D), w2, b2.reshape(1, D),
      g.reshape(1, D), bt.reshape(1, D))


def _edge_body(g1_ref, g2_ref, ea_ref, wc_ref, b1_ref, w2_ref, b2_ref,
               g_ref, bt_ref, out_ref):
    ea = ea_ref[...]
    z = (g1_ref[...] + g2_ref[...] + b1_ref[...]
         + jnp.dot(ea, wc_ref[...], preferred_element_type=jnp.float32))
    h = _silu(z)
    h = _silu(jnp.dot(h, w2_ref[...],
                      preferred_element_type=jnp.float32) + b2_ref[...])
    out_ref[...] = _ln(h, g_ref[...], bt_ref[...]) + ea


def _edge_mlp(g1, g2, ea, wc, b1, w2, b2, g, bt):
    e = ea.shape[0]
    blk = pl.BlockSpec((BE, D), lambda i: (i, 0))
    return pl.pallas_call(
        _edge_body,
        grid=(e // BE,),
        in_specs=[blk, blk, blk, _full((D, D)), _full((1, D)),
                  _full((D, D)), _full((1, D)), _full((1, D)), _full((1, D))],
        out_specs=blk,
        out_shape=jax.ShapeDtypeStruct((e, D), jnp.float32),
    )(g1, g2, ea, wc, b1.reshape(1, D), w2, b2.reshape(1, D),
      g.reshape(1, D), bt.reshape(1, D))


def _node_body(h_ref, a0_ref, a1_ref, a2_ref, a3_ref, w1a_ref, w1b_ref,
               b1_ref, w2_ref, b2_ref, g_ref, bt_ref, wa_ref, wb_ref,
               out_ref, p_ref, q_ref):
    h = h_ref[...]
    a = (a0_ref[0] + a1_ref[0]) + (a2_ref[0] + a3_ref[0])
    z = (jnp.dot(h, w1a_ref[...], preferred_element_type=jnp.float32)
         + jnp.dot(a, w1b_ref[...], preferred_element_type=jnp.float32)
         + b1_ref[...])
    t = _silu(z)
    t = _silu(jnp.dot(t, w2_ref[...],
                      preferred_element_type=jnp.float32) + b2_ref[...])
    hn = _ln(t, g_ref[...], bt_ref[...]) + h
    out_ref[...] = hn
    p_ref[...] = jnp.dot(hn, wa_ref[...], preferred_element_type=jnp.float32)
    q_ref[...] = jnp.dot(hn, wb_ref[...], preferred_element_type=jnp.float32)


def _node_mlp(h, aggpA, aggpB, w1, b1, w2, b2, g, bt, wa, wb):
    n = h.shape[0]
    blk = pl.BlockSpec((BN, D), lambda i: (i, 0))
    a0 = pl.BlockSpec((1, BN, D), lambda i: (0, i, 0))
    a1 = pl.BlockSpec((1, BN, D), lambda i: (1, i, 0))
    return pl.pallas_call(
        _node_body,
        grid=(n // BN,),
        in_specs=[blk, a0, a1, a0, a1, _full((D, D)), _full((D, D)),
                  _full((1, D)), _full((D, D)), _full((1, D)), _full((1, D)),
                  _full((1, D)), _full((D, D)), _full((D, D))],
        out_specs=(blk, blk, blk),
        out_shape=(jax.ShapeDtypeStruct((n, D), jnp.float32),
                   jax.ShapeDtypeStruct((n, D), jnp.float32),
                   jax.ShapeDtypeStruct((n, D), jnp.float32)),
    )(h, aggpA, aggpA, aggpB, aggpB, w1[:D], w1[D:], b1.reshape(1, D), w2,
      b2.reshape(1, D), g.reshape(1, D), bt.reshape(1, D), wa, wb)


def _proj_body(h_ref, wa_ref, wb_ref, p_ref, q_ref):
    h = h_ref[...]
    p_ref[...] = jnp.dot(h, wa_ref[...], preferred_element_type=jnp.float32)
    q_ref[...] = jnp.dot(h, wb_ref[...], preferred_element_type=jnp.float32)


def _proj(h, wa, wb):
    n = h.shape[0]
    blk = pl.BlockSpec((BN, D), lambda i: (i, 0))
    return pl.pallas_call(
        _proj_body,
        grid=(n // BN,),
        in_specs=[blk, _full((D, D)), _full((D, D))],
        out_specs=(blk, blk),
        out_shape=(jax.ShapeDtypeStruct((n, D), jnp.float32),
                   jax.ShapeDtypeStruct((n, D), jnp.float32)),
    )(h, wa, wb)


def _sc_gather(p, q, dst, src):
    n, d = p.shape
    e = dst.shape[0]
    ew = e // NW
    steps = (ew + CB - 1) // CB
    steps = ((steps + 2) // 3) * 3
    last_base = ew - CB
    NB = 3

    mesh = plsc.VectorSubcoreMesh(core_axis_name="c", subcore_axis_name="s", num_cores=NC, num_subcores=NS)

    @functools.partial(
        pl.kernel, mesh=mesh,
        out_type=(jax.ShapeDtypeStruct((e, d), jnp.float32),
                  jax.ShapeDtypeStruct((e, d), jnp.float32)),
        scratch_types=[
            pltpu.VMEM((ew,), jnp.int32),
            pltpu.VMEM((ew,), jnp.int32),
            pltpu.VMEM((NB, CB, d), jnp.float32),
            pltpu.VMEM((NB, CB, d), jnp.float32),
        ] + [pltpu.SemaphoreType.DMA] * (4 * NB),
    )
    def k(p_hbm, q_hbm, dst_hbm, src_hbm, g1_hbm, g2_hbm,
          idxd, idxs, bufp, bufq, *sems):
        semp, semq, stp, stq = (sems[0:NB], sems[NB:2 * NB],
                                sems[2 * NB:3 * NB], sems[3 * NB:4 * NB])
        w = lax.axis_index("s") * NC + lax.axis_index("c")
        base0 = pl.multiple_of(w * ew, 8)
        pltpu.sync_copy(dst_hbm.at[pl.ds(base0, ew)], idxd)
        pltpu.sync_copy(src_hbm.at[pl.ds(base0, ew)], idxs)

        def cbase(i):
            return pl.multiple_of(jnp.minimum(i * CB, last_base), 8)

        def start(i, slot):
            cb = cbase(i)
            pltpu.async_copy(p_hbm.at[idxd.at[pl.ds(cb, CB)]],
                             bufp.at[slot], semp[slot])
            pltpu.async_copy(q_hbm.at[idxs.at[pl.ds(cb, CB)]],
                             bufq.at[slot], semq[slot])

        def wait_gathers(i, slot):
            cb = cbase(i)
            pltpu.make_async_copy(p_hbm.at[idxd.at[pl.ds(cb, CB)]],
                                  bufp.at[slot], semp[slot]).wait()
            pltpu.make_async_copy(q_hbm.at[idxs.at[pl.ds(cb, CB)]],
                                  bufq.at[slot], semq[slot]).wait()

        def start_stores(i, slot):
            cb = cbase(i)
            pltpu.async_copy(bufp.at[slot], g1_hbm.at[pl.ds(base0 + cb, CB)],
                             stp[slot])
            pltpu.async_copy(bufq.at[slot], g2_hbm.at[pl.ds(base0 + cb, CB)],
                             stq[slot])

        def wait_stores(i, slot):
            cb = cbase(i)
            pltpu.make_async_copy(bufp.at[slot], g1_hbm.at[pl.ds(base0 + cb, CB)],
                                  stp[slot]).wait()
            pltpu.make_async_copy(bufq.at[slot], g2_hbm.at[pl.ds(base0 + cb, CB)],
                                  stq[slot]).wait()

        start(0, 0)
        start(1, 1)

        def triple(qq, carry):
            i0 = qq * NB
            for u in range(NB):
                i = i0 + u
                nxt = (u + 2) % NB

                @pl.when(i + 2 < steps)
                def _():
                    @pl.when(i >= 1)
                    def _():
                        wait_stores(i - 1, nxt)

                    start(i + 2, nxt)

                wait_gathers(i, u)
                start_stores(i, u)
            return carry

        lax.fori_loop(0, steps // NB, triple, 0)
        for u in range(NB):
            wait_stores(steps - NB + u, (steps - NB + u) % NB)

    return k(p, q, dst, src)


def _sc_scatter(m, dst1, zblk, n):
    e, d = m.shape
    nchunks = e // CB
    g8, rem = divmod(nchunks, 8)
    b8, x = divmod(g8, NW)
    assert rem % 2 == 0
    big, small = 8 * (b8 + 1), 8 * b8
    win = max(big if x else small, small + rem)
    rows_needed = small * (NW - 1) + 8 * min(NW - 1, x) + win
    dst2 = jnp.concatenate(
        [dst1, jnp.zeros((rows_needed * CB - e,), jnp.int32)]
    ).reshape(rows_needed, CB)
    n_pad = ((n + 16 * CB - 1) // (16 * CB)) * (16 * CB)
    zrows = n_pad // NS

    mesh = plsc.VectorSubcoreMesh(core_axis_name="c", subcore_axis_name="s", num_cores=NC, num_subcores=NS)

    @functools.partial(
        pl.kernel, mesh=mesh,
        out_type=jax.ShapeDtypeStruct((NC, n_pad, d), jnp.float32),
        scratch_types=[
            pltpu.VMEM((win, CB), jnp.int32),
            pltpu.VMEM((2, CB, d), jnp.float32),
            pltpu.VMEM_SHARED((n_pad, d), jnp.float32),
            pltpu.SemaphoreType.DMA,
            pltpu.SemaphoreType.DMA,
        ],
    )
    def k(m_hbm, dst2_hbm, zblk_hbm, out_hbm, idx, rowbuf, acc, sem0, sem1):
        c = lax.axis_index("c")
        s = lax.axis_index("s")
        w = s * NC + c
        ncw = (jnp.where(w < x, big, small)
               + jnp.where(w == NW - 1, rem, 0))
        cstart = pl.multiple_of(small * w + 8 * jnp.minimum(w, x), 8)
        pltpu.sync_copy(dst2_hbm.at[pl.ds(cstart, win)], idx)
        for r in range(zrows // CB):
            pltpu.sync_copy(zblk_hbm, acc.at[pl.ds(s * zrows + r * CB, CB)])
        plsc.subcore_barrier()
        sems = (sem0, sem1)

        def start(j, slot):
            base = pl.multiple_of((cstart + j) * CB, 8)
            pltpu.async_copy(m_hbm.at[pl.ds(base, CB)], rowbuf.at[slot],
                             sems[slot])

        def finish(j, slot):
            base = pl.multiple_of((cstart + j) * CB, 8)
            pltpu.make_async_copy(m_hbm.at[pl.ds(base, CB)], rowbuf.at[slot],
                                  sems[slot]).wait()
            pltpu.sync_copy(rowbuf.at[slot], acc.at[idx.at[j]], add=True)

        start(0, 0)

        def pair(ii, carry):
            j0 = ii * 2
            start(j0 + 1, 1)
            finish(j0, 0)

            @pl.when(j0 + 2 < ncw)
            def _():
                start(j0 + 2, 0)

            finish(j0 + 1, 1)
            return carry

        lax.fori_loop(0, ncw // 2, pair, 0)
        plsc.subcore_barrier()
        pltpu.sync_copy(acc.at[pl.ds(s * zrows, zrows)],
                        out_hbm.at[c, pl.ds(s * zrows, zrows)])

    return k(m, dst2, zblk)


def kernel(x, edge_index, edge_attr, emb_W1, emb_b1, emb_W2, emb_b2, emb_g,
           emb_bt, eW1, eb1, eW2, eb2, eg, ebt, nW1, nb1, nW2, nb2, ng, nbt):
    src = edge_index[0]
    dst = edge_index[1]
    n = x.shape[0]
    e = dst.shape[0]
    zblk = jnp.zeros((CB, D), jnp.float32)

    e2 = e // 2
    halves = []
    for hx in range(2):
        sl = slice(hx * e2, (hx + 1) * e2)
        halves.append({
            "dst": dst[sl], "src": src[sl],
            "ea": _emb_mlp(edge_attr, emb_W1, emb_b1, emb_W2, emb_b2, emb_g,
                           emb_bt, hx * (e2 // BE), e2 // BE),
        })

    h = x
    num_layers = eW1.shape[0]
    zw = jnp.zeros((D, D), jnp.float32)
    p, q = _proj(x, eW1[0, :D], eW1[0, D:2 * D])
    for l in range(num_layers):
        wc = eW1[l, 2 * D:]
        aggs = []
        for hv in halves:
            g1, g2 = _sc_gather(p, q, hv["dst"], hv["src"])
            m = _edge_mlp(g1, g2, hv["ea"], wc, eb1[l], eW2[l], eb2[l],
                          eg[l], ebt[l])
            aggs.append(_sc_scatter(m, hv["dst"], zblk, n))
            hv["ea"] = m
        last = l == num_layers - 1
        wa_n = zw if last else eW1[l + 1, :D]
        wb_n = zw if last else eW1[l + 1, D:2 * D]
        h, p, q = _node_mlp(h, aggs[0], aggs[1], nW1[l], nb1[l], nW2[l],
                            nb2[l], ng[l], nbt[l], wa_n, wb_n)
    return h


# speedup vs baseline: 1.0169x; 1.0169x over previous
import functools

import jax
import jax.numpy as jnp
from jax import lax
from jax.experimental import pallas as pl
from jax.experimental.pallas import tpu as pltpu
from jax.experimental.pallas import tpu_sc as plsc

D = 128
BE = 4000
BN = 2000

NC = 2
NS = 16
NW = NC * NS
CB = 128


def _silu(v):
    return v * jax.nn.sigmoid(v)


def _ln(v, g, b, eps=1e-5):
    m = jnp.mean(v, axis=-1, keepdims=True)
    c = v - m
    var = jnp.mean(c * c, axis=-1, keepdims=True)
    return c * jax.lax.rsqrt(var + eps) * g + b


def _full(shape):
    return pl.BlockSpec(shape, lambda i: tuple(0 for _ in shape))


def _emb_body(ea_ref, w1_ref, b1_ref, w2_ref, b2_ref, g_ref, bt_ref, out_ref):
    h = _silu(jnp.dot(ea_ref[...], w1_ref[...],
                      preferred_element_type=jnp.float32) + b1_ref[...])
    h = _silu(jnp.dot(h, w2_ref[...],
                      preferred_element_type=jnp.float32) + b2_ref[...])
    out_ref[...] = _ln(h, g_ref[...], bt_ref[...])


def _emb_mlp(ea, w1, b1, w2, b2, g, bt, off_blocks, nblocks):
    ed = ea.shape[1]
    return pl.pallas_call(
        _emb_body,
        grid=(nblocks,),
        in_specs=[
            pl.BlockSpec((BE, ed), lambda i: (i + off_blocks, 0)),
            _full((ed, D)), _full((1, D)), _full((D, D)), _full((1, D)),
            _full((1, D)), _full((1, D)),
        ],
        out_specs=pl.BlockSpec((BE, D), lambda i: (i, 0)),
        out_shape=jax.ShapeDtypeStruct((nblocks * BE, D), jnp.float32),
    )(ea, w1, b1.reshape(1, D), w2, b2.reshape(1, D),
      g.reshape(1, D), bt.reshape(1, D))


def _edge_body(g1_ref, g2_ref, ea_ref, wc_ref, b1_ref, w2_ref, b2_ref,
               g_ref, bt_ref, out_ref):
    ea = ea_ref[...]
    z = (g1_ref[...] + g2_ref[...] + b1_ref[...]
         + jnp.dot(ea, wc_ref[...], preferred_element_type=jnp.float32))
    h = _silu(z)
    h = _silu(jnp.dot(h, w2_ref[...],
                      preferred_element_type=jnp.float32) + b2_ref[...])
    out_ref[...] = _ln(h, g_ref[...], bt_ref[...]) + ea


def _edge_mlp(g1, g2, ea, wc, b1, w2, b2, g, bt):
    e = ea.shape[0]
    blk = pl.BlockSpec((BE, D), lambda i: (i, 0))
    return pl.pallas_call(
        _edge_body,
        grid=(e // BE,),
        in_specs=[blk, blk, blk, _full((D, D)), _full((1, D)),
                  _full((D, D)), _full((1, D)), _full((1, D)), _full((1, D))],
        out_specs=blk,
        out_shape=jax.ShapeDtypeStruct((e, D), jnp.float32),
    )(g1, g2, ea, wc, b1.reshape(1, D), w2, b2.reshape(1, D),
      g.reshape(1, D), bt.reshape(1, D))


def _node_body(h_ref, a0_ref, a1_ref, a2_ref, a3_ref, w1a_ref, w1b_ref,
               b1_ref, w2_ref, b2_ref, g_ref, bt_ref, wa_ref, wb_ref,
               out_ref, p_ref, q_ref):
    h = h_ref[...]
    a = (a0_ref[0] + a1_ref[0]) + (a2_ref[0] + a3_ref[0])
    z = (jnp.dot(h, w1a_ref[...], preferred_element_type=jnp.float32)
         + jnp.dot(a, w1b_ref[...], preferred_element_type=jnp.float32)
         + b1_ref[...])
    t = _silu(z)
    t = _silu(jnp.dot(t, w2_ref[...],
                      preferred_element_type=jnp.float32) + b2_ref[...])
    hn = _ln(t, g_ref[...], bt_ref[...]) + h
    out_ref[...] = hn
    p_ref[...] = jnp.dot(hn, wa_ref[...], preferred_element_type=jnp.float32)
    q_ref[...] = jnp.dot(hn, wb_ref[...], preferred_element_type=jnp.float32)


def _node_mlp(h, aggpA, aggpB, w1, b1, w2, b2, g, bt, wa, wb):
    n = h.shape[0]
    blk = pl.BlockSpec((BN, D), lambda i: (i, 0))
    a0 = pl.BlockSpec((1, BN, D), lambda i: (0, i, 0))
    a1 = pl.BlockSpec((1, BN, D), lambda i: (1, i, 0))
    return pl.pallas_call(
        _node_body,
        grid=(n // BN,),
        in_specs=[blk, a0, a1, a0, a1, _full((D, D)), _full((D, D)),
                  _full((1, D)), _full((D, D)), _full((1, D)), _full((1, D)),
                  _full((1, D)), _full((D, D)), _full((D, D))],
        out_specs=(blk, blk, blk),
        out_shape=(jax.ShapeDtypeStruct((n, D), jnp.float32),
                   jax.ShapeDtypeStruct((n, D), jnp.float32),
                   jax.ShapeDtypeStruct((n, D), jnp.float32)),
    )(h, aggpA, aggpA, aggpB, aggpB, w1[:D], w1[D:], b1.reshape(1, D), w2,
      b2.reshape(1, D), g.reshape(1, D), bt.reshape(1, D), wa, wb)


def _proj_body(h_ref, wa_ref, wb_ref, p_ref, q_ref):
    h = h_ref[...]
    p_ref[...] = jnp.dot(h, wa_ref[...], preferred_element_type=jnp.float32)
    q_ref[...] = jnp.dot(h, wb_ref[...], preferred_element_type=jnp.float32)


def _proj(h, wa, wb):
    n = h.shape[0]
    blk = pl.BlockSpec((BN, D), lambda i: (i, 0))
    return pl.pallas_call(
        _proj_body,
        grid=(n // BN,),
        in_specs=[blk, _full((D, D)), _full((D, D))],
        out_specs=(blk, blk),
        out_shape=(jax.ShapeDtypeStruct((n, D), jnp.float32),
                   jax.ShapeDtypeStruct((n, D), jnp.float32)),
    )(h, wa, wb)


def _sc_gather(p, q, dst, src):
    n, d = p.shape
    e = dst.shape[0]
    ew = e // NW
    steps = (ew + CB - 1) // CB
    steps += steps % 2
    last_base = ew - CB

    mesh = plsc.VectorSubcoreMesh(core_axis_name="c", subcore_axis_name="s", num_cores=NC, num_subcores=NS)

    @functools.partial(
        pl.kernel, mesh=mesh,
        out_type=(jax.ShapeDtypeStruct((e, d), jnp.float32),
                  jax.ShapeDtypeStruct((e, d), jnp.float32)),
        scratch_types=[
            pltpu.VMEM((ew,), jnp.int32),
            pltpu.VMEM((ew,), jnp.int32),
            pltpu.VMEM((2, CB, d), jnp.float32),
            pltpu.VMEM((2, CB, d), jnp.float32),
            pltpu.SemaphoreType.DMA,
            pltpu.SemaphoreType.DMA,
            pltpu.SemaphoreType.DMA,
            pltpu.SemaphoreType.DMA,
        ],
    )
    def k(p_hbm, q_hbm, dst_hbm, src_hbm, g1_hbm, g2_hbm,
          idxd, idxs, bufp, bufq, semp0, semp1, semq0, semq1):
        w = lax.axis_index("s") * NC + lax.axis_index("c")
        base0 = pl.multiple_of(w * ew, 8)
        pltpu.sync_copy(dst_hbm.at[pl.ds(base0, ew)], idxd)
        pltpu.sync_copy(src_hbm.at[pl.ds(base0, ew)], idxs)
        semp = (semp0, semp1)
        semq = (semq0, semq1)

        def cbase(i):
            return pl.multiple_of(jnp.minimum(i * CB, last_base), 8)

        def start(i, slot):
            cb = cbase(i)
            pltpu.async_copy(p_hbm.at[idxd.at[pl.ds(cb, CB)]],
                             bufp.at[slot], semp[slot])
            pltpu.async_copy(q_hbm.at[idxs.at[pl.ds(cb, CB)]],
                             bufq.at[slot], semq[slot])

        def finish(i, slot):
            cb = cbase(i)
            pltpu.make_async_copy(p_hbm.at[idxd.at[pl.ds(cb, CB)]],
                                  bufp.at[slot], semp[slot]).wait()
            pltpu.make_async_copy(q_hbm.at[idxs.at[pl.ds(cb, CB)]],
                                  bufq.at[slot], semq[slot]).wait()
            pltpu.sync_copy(bufp.at[slot], g1_hbm.at[pl.ds(base0 + cb, CB)])
            pltpu.sync_copy(bufq.at[slot], g2_hbm.at[pl.ds(base0 + cb, CB)])

        start(0, 0)

        def pair(ii, carry):
            i0 = ii * 2
            start(i0 + 1, 1)
            finish(i0, 0)

            @pl.when(i0 + 2 < steps)
            def _():
                start(i0 + 2, 0)

            finish(i0 + 1, 1)
            return carry

        lax.fori_loop(0, steps // 2, pair, 0)

    return k(p, q, dst, src)


def _sc_scatter(m, dst1, zblk, n):
    e, d = m.shape
    nchunks = e // CB
    g8, rem = divmod(nchunks, 8)
    b8, x = divmod(g8, NW)
    assert rem % 2 == 0
    big, small = 8 * (b8 + 1), 8 * b8
    win = max(big if x else small, small + rem)
    rows_needed = small * (NW - 1) + 8 * min(NW - 1, x) + win
    dst2 = jnp.concatenate(
        [dst1, jnp.zeros((rows_needed * CB - e,), jnp.int32)]
    ).reshape(rows_needed, CB)
    n_pad = ((n + 16 * CB - 1) // (16 * CB)) * (16 * CB)
    zrows = n_pad // NS

    mesh = plsc.VectorSubcoreMesh(core_axis_name="c", subcore_axis_name="s", num_cores=NC, num_subcores=NS)

    @functools.partial(
        pl.kernel, mesh=mesh,
        out_type=jax.ShapeDtypeStruct((NC, n_pad, d), jnp.float32),
        scratch_types=[
            pltpu.VMEM((win, CB), jnp.int32),
            pltpu.VMEM((2, CB, d), jnp.float32),
            pltpu.VMEM_SHARED((n_pad, d), jnp.float32),
            pltpu.SemaphoreType.DMA,
            pltpu.SemaphoreType.DMA,
        ],
    )
    def k(m_hbm, dst2_hbm, zblk_hbm, out_hbm, idx, rowbuf, acc, sem0, sem1):
        c = lax.axis_index("c")
        s = lax.axis_index("s")
        w = s * NC + c
        ncw = (jnp.where(w < x, big, small)
               + jnp.where(w == NW - 1, rem, 0))
        cstart = pl.multiple_of(small * w + 8 * jnp.minimum(w, x), 8)
        pltpu.sync_copy(dst2_hbm.at[pl.ds(cstart, win)], idx)
        for r in range(zrows // CB):
            pltpu.sync_copy(zblk_hbm, acc.at[pl.ds(s * zrows + r * CB, CB)])
        plsc.subcore_barrier()
        sems = (sem0, sem1)

        def start(j, slot):
            base = pl.multiple_of((cstart + j) * CB, 8)
            pltpu.async_copy(m_hbm.at[pl.ds(base, CB)], rowbuf.at[slot],
                             sems[slot])

        def finish(j, slot):
            base = pl.multiple_of((cstart + j) * CB, 8)
            pltpu.make_async_copy(m_hbm.at[pl.ds(base, CB)], rowbuf.at[slot],
                                  sems[slot]).wait()
            pltpu.sync_copy(rowbuf.at[slot], acc.at[idx.at[j]], add=True)

        start(0, 0)

        def pair(ii, carry):
            j0 = ii * 2
            start(j0 + 1, 1)
            finish(j0, 0)

            @pl.when(j0 + 2 < ncw)
            def _():
                start(j0 + 2, 0)

            finish(j0 + 1, 1)
            return carry

        lax.fori_loop(0, ncw // 2, pair, 0)
        plsc.subcore_barrier()
        pltpu.sync_copy(acc.at[pl.ds(s * zrows, zrows)],
                        out_hbm.at[c, pl.ds(s * zrows, zrows)])

    return k(m, dst2, zblk)


def kernel(x, edge_index, edge_attr, emb_W1, emb_b1, emb_W2, emb_b2, emb_g,
           emb_bt, eW1, eb1, eW2, eb2, eg, ebt, nW1, nb1, nW2, nb2, ng, nbt):
    src = edge_index[0]
    dst = edge_index[1]
    n = x.shape[0]
    e = dst.shape[0]
    zblk = jnp.zeros((CB, D), jnp.float32)

    e2 = e // 2
    halves = []
    for hx in range(2):
        sl = slice(hx * e2, (hx + 1) * e2)
        halves.append({
            "dst": dst[sl], "src": src[sl],
            "ea": _emb_mlp(edge_attr, emb_W1, emb_b1, emb_W2, emb_b2, emb_g,
                           emb_bt, hx * (e2 // BE), e2 // BE),
        })

    h = x
    num_layers = eW1.shape[0]
    zw = jnp.zeros((D, D), jnp.float32)
    p, q = _proj(x, eW1[0, :D], eW1[0, D:2 * D])
    for l in range(num_layers):
        wc = eW1[l, 2 * D:]
        aggs = []
        for hv in halves:
            g1, g2 = _sc_gather(p, q, hv["dst"], hv["src"])
            m = _edge_mlp(g1, g2, hv["ea"], wc, eb1[l], eW2[l], eb2[l],
                          eg[l], ebt[l])
            aggs.append(_sc_scatter(m, hv["dst"], zblk, n))
            hv["ea"] = m
        last = l == num_layers - 1
        wa_n = zw if last else eW1[l + 1, :D]
        wb_n = zw if last else eW1[l + 1, D:2 * D]
        h, p, q = _node_mlp(h, aggs[0], aggs[1], nW1[l], nb1[l], nW2[l],
                            nb2[l], ng[l], nbt[l], wa_n, wb_n)
    return h


# speedup vs baseline: 1.0302x; 1.0130x over previous
import functools

import jax
import jax.numpy as jnp
from jax import lax
from jax.experimental import pallas as pl
from jax.experimental.pallas import tpu as pltpu
from jax.experimental.pallas import tpu_sc as plsc

D = 128
BE = 8000
BN = 2000

NC = 2
NS = 16
NW = NC * NS
CB = 128


def _silu(v):
    return v * jax.nn.sigmoid(v)


def _ln(v, g, b, eps=1e-5):
    m = jnp.mean(v, axis=-1, keepdims=True)
    c = v - m
    var = jnp.mean(c * c, axis=-1, keepdims=True)
    return c * jax.lax.rsqrt(var + eps) * g + b


def _full(shape):
    return pl.BlockSpec(shape, lambda i: tuple(0 for _ in shape))


def _emb_body(ea_ref, w1_ref, b1_ref, w2_ref, b2_ref, g_ref, bt_ref, out_ref):
    h = _silu(jnp.dot(ea_ref[...], w1_ref[...],
                      preferred_element_type=jnp.float32) + b1_ref[...])
    h = _silu(jnp.dot(h, w2_ref[...],
                      preferred_element_type=jnp.float32) + b2_ref[...])
    out_ref[...] = _ln(h, g_ref[...], bt_ref[...])


def _emb_mlp(ea, w1, b1, w2, b2, g, bt, off_blocks, nblocks):
    ed = ea.shape[1]
    return pl.pallas_call(
        _emb_body,
        grid=(nblocks,),
        in_specs=[
            pl.BlockSpec((BE, ed), lambda i: (i + off_blocks, 0)),
            _full((ed, D)), _full((1, D)), _full((D, D)), _full((1, D)),
            _full((1, D)), _full((1, D)),
        ],
        out_specs=pl.BlockSpec((BE, D), lambda i: (i, 0)),
        out_shape=jax.ShapeDtypeStruct((nblocks * BE, D), jnp.float32),
    )(ea, w1, b1.reshape(1, D), w2, b2.reshape(1, D),
      g.reshape(1, D), bt.reshape(1, D))


def _edge_body(g1_ref, g2_ref, ea_ref, wc_ref, b1_ref, w2_ref, b2_ref,
               g_ref, bt_ref, out_ref):
    ea = ea_ref[...]
    z = (g1_ref[...] + g2_ref[...] + b1_ref[...]
         + jnp.dot(ea, wc_ref[...], preferred_element_type=jnp.float32))
    h = _silu(z)
    h = _silu(jnp.dot(h, w2_ref[...],
                      preferred_element_type=jnp.float32) + b2_ref[...])
    out_ref[...] = _ln(h, g_ref[...], bt_ref[...]) + ea


def _edge_mlp(g1, g2, ea, wc, b1, w2, b2, g, bt):
    e = ea.shape[0]
    blk = pl.BlockSpec((BE, D), lambda i: (i, 0))
    return pl.pallas_call(
        _edge_body,
        grid=(e // BE,),
        in_specs=[blk, blk, blk, _full((D, D)), _full((1, D)),
                  _full((D, D)), _full((1, D)), _full((1, D)), _full((1, D))],
        out_specs=blk,
        out_shape=jax.ShapeDtypeStruct((e, D), jnp.float32),
    )(g1, g2, ea, wc, b1.reshape(1, D), w2, b2.reshape(1, D),
      g.reshape(1, D), bt.reshape(1, D))


def _node_body(h_ref, a0_ref, a1_ref, a2_ref, a3_ref, w1a_ref, w1b_ref,
               b1_ref, w2_ref, b2_ref, g_ref, bt_ref, wa_ref, wb_ref,
               out_ref, p_ref, q_ref):
    h = h_ref[...]
    a = (a0_ref[0] + a1_ref[0]) + (a2_ref[0] + a3_ref[0])
    z = (jnp.dot(h, w1a_ref[...], preferred_element_type=jnp.float32)
         + jnp.dot(a, w1b_ref[...], preferred_element_type=jnp.float32)
         + b1_ref[...])
    t = _silu(z)
    t = _silu(jnp.dot(t, w2_ref[...],
                      preferred_element_type=jnp.float32) + b2_ref[...])
    hn = _ln(t, g_ref[...], bt_ref[...]) + h
    out_ref[...] = hn
    p_ref[...] = jnp.dot(hn, wa_ref[...], preferred_element_type=jnp.float32)
    q_ref[...] = jnp.dot(hn, wb_ref[...], preferred_element_type=jnp.float32)


def _node_mlp(h, aggpA, aggpB, w1, b1, w2, b2, g, bt, wa, wb):
    n = h.shape[0]
    blk = pl.BlockSpec((BN, D), lambda i: (i, 0))
    a0 = pl.BlockSpec((1, BN, D), lambda i: (0, i, 0))
    a1 = pl.BlockSpec((1, BN, D), lambda i: (1, i, 0))
    return pl.pallas_call(
        _node_body,
        grid=(n // BN,),
        in_specs=[blk, a0, a1, a0, a1, _full((D, D)), _full((D, D)),
                  _full((1, D)), _full((D, D)), _full((1, D)), _full((1, D)),
                  _full((1, D)), _full((D, D)), _full((D, D))],
        out_specs=(blk, blk, blk),
        out_shape=(jax.ShapeDtypeStruct((n, D), jnp.float32),
                   jax.ShapeDtypeStruct((n, D), jnp.float32),
                   jax.ShapeDtypeStruct((n, D), jnp.float32)),
    )(h, aggpA, aggpA, aggpB, aggpB, w1[:D], w1[D:], b1.reshape(1, D), w2,
      b2.reshape(1, D), g.reshape(1, D), bt.reshape(1, D), wa, wb)


def _proj_body(h_ref, wa_ref, wb_ref, p_ref, q_ref):
    h = h_ref[...]
    p_ref[...] = jnp.dot(h, wa_ref[...], preferred_element_type=jnp.float32)
    q_ref[...] = jnp.dot(h, wb_ref[...], preferred_element_type=jnp.float32)


def _proj(h, wa, wb):
    n = h.shape[0]
    blk = pl.BlockSpec((BN, D), lambda i: (i, 0))
    return pl.pallas_call(
        _proj_body,
        grid=(n // BN,),
        in_specs=[blk, _full((D, D)), _full((D, D))],
        out_specs=(blk, blk),
        out_shape=(jax.ShapeDtypeStruct((n, D), jnp.float32),
                   jax.ShapeDtypeStruct((n, D), jnp.float32)),
    )(h, wa, wb)


def _sc_gather(p, q, dst, src):
    n, d = p.shape
    e = dst.shape[0]
    ew = e // NW
    steps = (ew + CB - 1) // CB
    steps += steps % 2
    last_base = ew - CB

    mesh = plsc.VectorSubcoreMesh(core_axis_name="c", subcore_axis_name="s", num_cores=NC, num_subcores=NS)

    @functools.partial(
        pl.kernel, mesh=mesh,
        out_type=(jax.ShapeDtypeStruct((e, d), jnp.float32),
                  jax.ShapeDtypeStruct((e, d), jnp.float32)),
        scratch_types=[
            pltpu.VMEM((ew,), jnp.int32),
            pltpu.VMEM((ew,), jnp.int32),
            pltpu.VMEM((2, CB, d), jnp.float32),
            pltpu.VMEM((2, CB, d), jnp.float32),
            pltpu.SemaphoreType.DMA,
            pltpu.SemaphoreType.DMA,
            pltpu.SemaphoreType.DMA,
            pltpu.SemaphoreType.DMA,
        ],
    )
    def k(p_hbm, q_hbm, dst_hbm, src_hbm, g1_hbm, g2_hbm,
          idxd, idxs, bufp, bufq, semp0, semp1, semq0, semq1):
        w = lax.axis_index("s") * NC + lax.axis_index("c")
        base0 = pl.multiple_of(w * ew, 8)
        pltpu.sync_copy(dst_hbm.at[pl.ds(base0, ew)], idxd)
        pltpu.sync_copy(src_hbm.at[pl.ds(base0, ew)], idxs)
        semp = (semp0, semp1)
        semq = (semq0, semq1)

        def cbase(i):
            return pl.multiple_of(jnp.minimum(i * CB, last_base), 8)

        def start(i, slot):
            cb = cbase(i)
            pltpu.async_copy(p_hbm.at[idxd.at[pl.ds(cb, CB)]],
                             bufp.at[slot], semp[slot])
            pltpu.async_copy(q_hbm.at[idxs.at[pl.ds(cb, CB)]],
                             bufq.at[slot], semq[slot])

        def finish(i, slot):
            cb = cbase(i)
            pltpu.make_async_copy(p_hbm.at[idxd.at[pl.ds(cb, CB)]],
                                  bufp.at[slot], semp[slot]).wait()
            pltpu.make_async_copy(q_hbm.at[idxs.at[pl.ds(cb, CB)]],
                                  bufq.at[slot], semq[slot]).wait()
            pltpu.sync_copy(bufp.at[slot], g1_hbm.at[pl.ds(base0 + cb, CB)])
            pltpu.sync_copy(bufq.at[slot], g2_hbm.at[pl.ds(base0 + cb, CB)])

        start(0, 0)

        def pair(ii, carry):
            i0 = ii * 2
            start(i0 + 1, 1)
            finish(i0, 0)

            @pl.when(i0 + 2 < steps)
            def _():
                start(i0 + 2, 0)

            finish(i0 + 1, 1)
            return carry

        lax.fori_loop(0, steps // 2, pair, 0)

    return k(p, q, dst, src)


def _sc_scatter(m, dst1, zblk, n):
    e, d = m.shape
    nchunks = e // CB
    g8, rem = divmod(nchunks, 8)
    b8, x = divmod(g8, NW)
    assert rem % 2 == 0
    big, small = 8 * (b8 + 1), 8 * b8
    win = max(big if x else small, small + rem)
    rows_needed = small * (NW - 1) + 8 * min(NW - 1, x) + win
    dst2 = jnp.concatenate(
        [dst1, jnp.zeros((rows_needed * CB - e,), jnp.int32)]
    ).reshape(rows_needed, CB)
    n_pad = ((n + 16 * CB - 1) // (16 * CB)) * (16 * CB)
    zrows = n_pad // NS

    mesh = plsc.VectorSubcoreMesh(core_axis_name="c", subcore_axis_name="s", num_cores=NC, num_subcores=NS)

    @functools.partial(
        pl.kernel, mesh=mesh,
        out_type=jax.ShapeDtypeStruct((NC, n_pad, d), jnp.float32),
        scratch_types=[
            pltpu.VMEM((win, CB), jnp.int32),
            pltpu.VMEM((2, CB, d), jnp.float32),
            pltpu.VMEM_SHARED((n_pad, d), jnp.float32),
            pltpu.SemaphoreType.DMA,
            pltpu.SemaphoreType.DMA,
        ],
    )
    def k(m_hbm, dst2_hbm, zblk_hbm, out_hbm, idx, rowbuf, acc, sem0, sem1):
        c = lax.axis_index("c")
        s = lax.axis_index("s")
        w = s * NC + c
        ncw = (jnp.where(w < x, big, small)
               + jnp.where(w == NW - 1, rem, 0))
        cstart = pl.multiple_of(small * w + 8 * jnp.minimum(w, x), 8)
        pltpu.sync_copy(dst2_hbm.at[pl.ds(cstart, win)], idx)
        for r in range(zrows // CB):
            pltpu.sync_copy(zblk_hbm, acc.at[pl.ds(s * zrows + r * CB, CB)])
        plsc.subcore_barrier()
        sems = (sem0, sem1)

        def start(j, slot):
            base = pl.multiple_of((cstart + j) * CB, 8)
            pltpu.async_copy(m_hbm.at[pl.ds(base, CB)], rowbuf.at[slot],
                             sems[slot])

        def finish(j, slot):
            base = pl.multiple_of((cstart + j) * CB, 8)
            pltpu.make_async_copy(m_hbm.at[pl.ds(base, CB)], rowbuf.at[slot],
                                  sems[slot]).wait()
            pltpu.sync_copy(rowbuf.at[slot], acc.at[idx.at[j]], add=True)

        start(0, 0)

        def pair(ii, carry):
            j0 = ii * 2
            start(j0 + 1, 1)
            finish(j0, 0)

            @pl.when(j0 + 2 < ncw)
            def _():
                start(j0 + 2, 0)

            finish(j0 + 1, 1)
            return carry

        lax.fori_loop(0, ncw // 2, pair, 0)
        plsc.subcore_barrier()
        pltpu.sync_copy(acc.at[pl.ds(s * zrows, zrows)],
                        out_hbm.at[c, pl.ds(s * zrows, zrows)])

    return k(m, dst2, zblk)


def kernel(x, edge_index, edge_attr, emb_W1, emb_b1, emb_W2, emb_b2, emb_g,
           emb_bt, eW1, eb1, eW2, eb2, eg, ebt, nW1, nb1, nW2, nb2, ng, nbt):
    src = edge_index[0]
    dst = edge_index[1]
    n = x.shape[0]
    e = dst.shape[0]
    zblk = jnp.zeros((CB, D), jnp.float32)

    e2 = e // 2
    halves = []
    for hx in range(2):
        sl = slice(hx * e2, (hx + 1) * e2)
        halves.append({
            "dst": dst[sl], "src": src[sl],
            "ea": _emb_mlp(edge_attr, emb_W1, emb_b1, emb_W2, emb_b2, emb_g,
                           emb_bt, hx * (e2 // BE), e2 // BE),
        })

    h = x
    num_layers = eW1.shape[0]
    zw = jnp.zeros((D, D), jnp.float32)
    p, q = _proj(x, eW1[0, :D], eW1[0, D:2 * D])
    for l in range(num_layers):
        wc = eW1[l, 2 * D:]
        aggs = []
        for hv in halves:
            g1, g2 = _sc_gather(p, q, hv["dst"], hv["src"])
            m = _edge_mlp(g1, g2, hv["ea"], wc, eb1[l], eW2[l], eb2[l],
                          eg[l], ebt[l])
            aggs.append(_sc_scatter(m, hv["dst"], zblk, n))
            hv["ea"] = m
        last = l == num_layers - 1
        wa_n = zw if last else eW1[l + 1, :D]
        wb_n = zw if last else eW1[l + 1, D:2 * D]
        h, p, q = _node_mlp(h, aggs[0], aggs[1], nW1[l], nb1[l], nW2[l],
                            nb2[l], ng[l], nbt[l], wa_n, wb_n)
    return h


# speedup vs baseline: 1.0315x; 1.0013x over previous
import functools

import jax
import jax.numpy as jnp
from jax import lax
from jax.experimental import pallas as pl
from jax.experimental.pallas import tpu as pltpu
from jax.experimental.pallas import tpu_sc as plsc

D = 128
BE = 10000
BN = 2000

NC = 2
NS = 16
NW = NC * NS
CB = 128


def _silu(v):
    return v * jax.nn.sigmoid(v)


def _ln(v, g, b, eps=1e-5):
    m = jnp.mean(v, axis=-1, keepdims=True)
    c = v - m
    var = jnp.mean(c * c, axis=-1, keepdims=True)
    return c * jax.lax.rsqrt(var + eps) * g + b


def _full(shape):
    return pl.BlockSpec(shape, lambda i: tuple(0 for _ in shape))


def _emb_body(ea_ref, w1_ref, b1_ref, w2_ref, b2_ref, g_ref, bt_ref, out_ref):
    h = _silu(jnp.dot(ea_ref[...], w1_ref[...],
                      preferred_element_type=jnp.float32) + b1_ref[...])
    h = _silu(jnp.dot(h, w2_ref[...],
                      preferred_element_type=jnp.float32) + b2_ref[...])
    out_ref[...] = _ln(h, g_ref[...], bt_ref[...])


def _emb_mlp(ea, w1, b1, w2, b2, g, bt, off_blocks, nblocks):
    ed = ea.shape[1]
    return pl.pallas_call(
        _emb_body,
        grid=(nblocks,),
        in_specs=[
            pl.BlockSpec((BE, ed), lambda i: (i + off_blocks, 0)),
            _full((ed, D)), _full((1, D)), _full((D, D)), _full((1, D)),
            _full((1, D)), _full((1, D)),
        ],
        out_specs=pl.BlockSpec((BE, D), lambda i: (i, 0)),
        out_shape=jax.ShapeDtypeStruct((nblocks * BE, D), jnp.float32),
    )(ea, w1, b1.reshape(1, D), w2, b2.reshape(1, D),
      g.reshape(1, D), bt.reshape(1, D))


def _edge_body(g1_ref, g2_ref, ea_ref, wc_ref, b1_ref, w2_ref, b2_ref,
               g_ref, bt_ref, out_ref):
    ea = ea_ref[...]
    z = (g1_ref[...] + g2_ref[...] + b1_ref[...]
         + jnp.dot(ea, wc_ref[...], preferred_element_type=jnp.float32))
    h = _silu(z)
    h = _silu(jnp.dot(h, w2_ref[...],
                      preferred_element_type=jnp.float32) + b2_ref[...])
    out_ref[...] = _ln(h, g_ref[...], bt_ref[...]) + ea


def _edge_mlp(g1, g2, ea, wc, b1, w2, b2, g, bt):
    e = ea.shape[0]
    blk = pl.BlockSpec((BE, D), lambda i: (i, 0))
    return pl.pallas_call(
        _edge_body,
        grid=(e // BE,),
        in_specs=[blk, blk, blk, _full((D, D)), _full((1, D)),
                  _full((D, D)), _full((1, D)), _full((1, D)), _full((1, D))],
        out_specs=blk,
        out_shape=jax.ShapeDtypeStruct((e, D), jnp.float32),
    )(g1, g2, ea, wc, b1.reshape(1, D), w2, b2.reshape(1, D),
      g.reshape(1, D), bt.reshape(1, D))


def _node_body(h_ref, a0_ref, a1_ref, a2_ref, a3_ref, w1a_ref, w1b_ref,
               b1_ref, w2_ref, b2_ref, g_ref, bt_ref, wa_ref, wb_ref,
               out_ref, p_ref, q_ref):
    h = h_ref[...]
    a = (a0_ref[0] + a1_ref[0]) + (a2_ref[0] + a3_ref[0])
    z = (jnp.dot(h, w1a_ref[...], preferred_element_type=jnp.float32)
         + jnp.dot(a, w1b_ref[...], preferred_element_type=jnp.float32)
         + b1_ref[...])
    t = _silu(z)
    t = _silu(jnp.dot(t, w2_ref[...],
                      preferred_element_type=jnp.float32) + b2_ref[...])
    hn = _ln(t, g_ref[...], bt_ref[...]) + h
    out_ref[...] = hn
    p_ref[...] = jnp.dot(hn, wa_ref[...], preferred_element_type=jnp.float32)
    q_ref[...] = jnp.dot(hn, wb_ref[...], preferred_element_type=jnp.float32)


def _node_mlp(h, aggpA, aggpB, w1, b1, w2, b2, g, bt, wa, wb):
    n = h.shape[0]
    blk = pl.BlockSpec((BN, D), lambda i: (i, 0))
    a0 = pl.BlockSpec((1, BN, D), lambda i: (0, i, 0))
    a1 = pl.BlockSpec((1, BN, D), lambda i: (1, i, 0))
    return pl.pallas_call(
        _node_body,
        grid=(n // BN,),
        in_specs=[blk, a0, a1, a0, a1, _full((D, D)), _full((D, D)),
                  _full((1, D)), _full((D, D)), _full((1, D)), _full((1, D)),
                  _full((1, D)), _full((D, D)), _full((D, D))],
        out_specs=(blk, blk, blk),
        out_shape=(jax.ShapeDtypeStruct((n, D), jnp.float32),
                   jax.ShapeDtypeStruct((n, D), jnp.float32),
                   jax.ShapeDtypeStruct((n, D), jnp.float32)),
    )(h, aggpA, aggpA, aggpB, aggpB, w1[:D], w1[D:], b1.reshape(1, D), w2,
      b2.reshape(1, D), g.reshape(1, D), bt.reshape(1, D), wa, wb)


def _proj_body(h_ref, wa_ref, wb_ref, p_ref, q_ref):
    h = h_ref[...]
    p_ref[...] = jnp.dot(h, wa_ref[...], preferred_element_type=jnp.float32)
    q_ref[...] = jnp.dot(h, wb_ref[...], preferred_element_type=jnp.float32)


def _proj(h, wa, wb):
    n = h.shape[0]
    blk = pl.BlockSpec((BN, D), lambda i: (i, 0))
    return pl.pallas_call(
        _proj_body,
        grid=(n // BN,),
        in_specs=[blk, _full((D, D)), _full((D, D))],
        out_specs=(blk, blk),
        out_shape=(jax.ShapeDtypeStruct((n, D), jnp.float32),
                   jax.ShapeDtypeStruct((n, D), jnp.float32)),
    )(h, wa, wb)


def _sc_gather(p, q, dst, src):
    n, d = p.shape
    e = dst.shape[0]
    ew = e // NW
    steps = (ew + CB - 1) // CB
    steps += steps % 2
    last_base = ew - CB

    mesh = plsc.VectorSubcoreMesh(core_axis_name="c", subcore_axis_name="s", num_cores=NC, num_subcores=NS)

    @functools.partial(
        pl.kernel, mesh=mesh,
        out_type=(jax.ShapeDtypeStruct((e, d), jnp.float32),
                  jax.ShapeDtypeStruct((e, d), jnp.float32)),
        scratch_types=[
            pltpu.VMEM((ew,), jnp.int32),
            pltpu.VMEM((ew,), jnp.int32),
            pltpu.VMEM((2, CB, d), jnp.float32),
            pltpu.VMEM((2, CB, d), jnp.float32),
            pltpu.SemaphoreType.DMA,
            pltpu.SemaphoreType.DMA,
            pltpu.SemaphoreType.DMA,
            pltpu.SemaphoreType.DMA,
        ],
    )
    def k(p_hbm, q_hbm, dst_hbm, src_hbm, g1_hbm, g2_hbm,
          idxd, idxs, bufp, bufq, semp0, semp1, semq0, semq1):
        w = lax.axis_index("s") * NC + lax.axis_index("c")
        base0 = pl.multiple_of(w * ew, 8)
        pltpu.sync_copy(dst_hbm.at[pl.ds(base0, ew)], idxd)
        pltpu.sync_copy(src_hbm.at[pl.ds(base0, ew)], idxs)
        semp = (semp0, semp1)
        semq = (semq0, semq1)

        def cbase(i):
            return pl.multiple_of(jnp.minimum(i * CB, last_base), 8)

        def start(i, slot):
            cb = cbase(i)
            pltpu.async_copy(p_hbm.at[idxd.at[pl.ds(cb, CB)]],
                             bufp.at[slot], semp[slot])
            pltpu.async_copy(q_hbm.at[idxs.at[pl.ds(cb, CB)]],
                             bufq.at[slot], semq[slot])

        def finish(i, slot):
            cb = cbase(i)
            pltpu.make_async_copy(p_hbm.at[idxd.at[pl.ds(cb, CB)]],
                                  bufp.at[slot], semp[slot]).wait()
            pltpu.make_async_copy(q_hbm.at[idxs.at[pl.ds(cb, CB)]],
                                  bufq.at[slot], semq[slot]).wait()
            pltpu.sync_copy(bufp.at[slot], g1_hbm.at[pl.ds(base0 + cb, CB)])
            pltpu.sync_copy(bufq.at[slot], g2_hbm.at[pl.ds(base0 + cb, CB)])

        start(0, 0)

        def pair(ii, carry):
            i0 = ii * 2
            start(i0 + 1, 1)
            finish(i0, 0)

            @pl.when(i0 + 2 < steps)
            def _():
                start(i0 + 2, 0)

            finish(i0 + 1, 1)
            return carry

        lax.fori_loop(0, steps // 2, pair, 0)

    return k(p, q, dst, src)


def _sc_scatter(m, dst1, zblk, n):
    e, d = m.shape
    nchunks = e // CB
    g8, rem = divmod(nchunks, 8)
    b8, x = divmod(g8, NW)
    assert rem % 2 == 0
    big, small = 8 * (b8 + 1), 8 * b8
    win = max(big if x else small, small + rem)
    rows_needed = small * (NW - 1) + 8 * min(NW - 1, x) + win
    dst2 = jnp.concatenate(
        [dst1, jnp.zeros((rows_needed * CB - e,), jnp.int32)]
    ).reshape(rows_needed, CB)
    n_pad = ((n + 16 * CB - 1) // (16 * CB)) * (16 * CB)
    zrows = n_pad // NS

    mesh = plsc.VectorSubcoreMesh(core_axis_name="c", subcore_axis_name="s", num_cores=NC, num_subcores=NS)

    @functools.partial(
        pl.kernel, mesh=mesh,
        out_type=jax.ShapeDtypeStruct((NC, n_pad, d), jnp.float32),
        scratch_types=[
            pltpu.VMEM((win, CB), jnp.int32),
            pltpu.VMEM((2, CB, d), jnp.float32),
            pltpu.VMEM_SHARED((n_pad, d), jnp.float32),
            pltpu.SemaphoreType.DMA,
            pltpu.SemaphoreType.DMA,
        ],
    )
    def k(m_hbm, dst2_hbm, zblk_hbm, out_hbm, idx, rowbuf, acc, sem0, sem1):
        c = lax.axis_index("c")
        s = lax.axis_index("s")
        w = s * NC + c
        ncw = (jnp.where(w < x, big, small)
               + jnp.where(w == NW - 1, rem, 0))
        cstart = pl.multiple_of(small * w + 8 * jnp.minimum(w, x), 8)
        pltpu.sync_copy(dst2_hbm.at[pl.ds(cstart, win)], idx)
        for r in range(zrows // CB):
            pltpu.sync_copy(zblk_hbm, acc.at[pl.ds(s * zrows + r * CB, CB)])
        plsc.subcore_barrier()
        sems = (sem0, sem1)

        def start(j, slot):
            base = pl.multiple_of((cstart + j) * CB, 8)
            pltpu.async_copy(m_hbm.at[pl.ds(base, CB)], rowbuf.at[slot],
                             sems[slot])

        def finish(j, slot):
            base = pl.multiple_of((cstart + j) * CB, 8)
            pltpu.make_async_copy(m_hbm.at[pl.ds(base, CB)], rowbuf.at[slot],
                                  sems[slot]).wait()
            pltpu.sync_copy(rowbuf.at[slot], acc.at[idx.at[j]], add=True)

        start(0, 0)

        def pair(ii, carry):
            j0 = ii * 2
            start(j0 + 1, 1)
            finish(j0, 0)

            @pl.when(j0 + 2 < ncw)
            def _():
                start(j0 + 2, 0)

            finish(j0 + 1, 1)
            return carry

        lax.fori_loop(0, ncw // 2, pair, 0)
        plsc.subcore_barrier()
        pltpu.sync_copy(acc.at[pl.ds(s * zrows, zrows)],
                        out_hbm.at[c, pl.ds(s * zrows, zrows)])

    return k(m, dst2, zblk)


def kernel(x, edge_index, edge_attr, emb_W1, emb_b1, emb_W2, emb_b2, emb_g,
           emb_bt, eW1, eb1, eW2, eb2, eg, ebt, nW1, nb1, nW2, nb2, ng, nbt):
    src = edge_index[0]
    dst = edge_index[1]
    n = x.shape[0]
    e = dst.shape[0]
    zblk = jnp.zeros((CB, D), jnp.float32)

    e2 = e // 2
    halves = []
    for hx in range(2):
        sl = slice(hx * e2, (hx + 1) * e2)
        halves.append({
            "dst": dst[sl], "src": src[sl],
            "ea": _emb_mlp(edge_attr, emb_W1, emb_b1, emb_W2, emb_b2, emb_g,
                           emb_bt, hx * (e2 // BE), e2 // BE),
        })

    h = x
    num_layers = eW1.shape[0]
    zw = jnp.zeros((D, D), jnp.float32)
    p, q = _proj(x, eW1[0, :D], eW1[0, D:2 * D])
    for l in range(num_layers):
        wc = eW1[l, 2 * D:]
        aggs = []
        for hv in halves:
            g1, g2 = _sc_gather(p, q, hv["dst"], hv["src"])
            m = _edge_mlp(g1, g2, hv["ea"], wc, eb1[l], eW2[l], eb2[l],
                          eg[l], ebt[l])
            aggs.append(_sc_scatter(m, hv["dst"], zblk, n))
            hv["ea"] = m
        last = l == num_layers - 1
        wa_n = zw if last else eW1[l + 1, :D]
        wb_n = zw if last else eW1[l + 1, D:2 * D]
        h, p, q = _node_mlp(h, aggs[0], aggs[1], nW1[l], nb1[l], nW2[l],
                            nb2[l], ng[l], nbt[l], wa_n, wb_n)
    return h
